# Initial kernel scaffold; baseline (speedup 1.0000x reference)
import functools

import jax
import jax.numpy as jnp
from jax import lax
from jax.experimental import pallas as pl
from jax.experimental.pallas import tpu as pltpu
from jax.experimental.pallas import tpu_sc as plsc

_INFO = plsc.get_sparse_core_info()
NC = _INFO.num_cores
NS = _INFO.num_subcores
NW = NC * NS
SUB = 80


def _gather_body(n_per_w, krow, table, ids2d, out, idx_v, rows_v, sem):
    cid = lax.axis_index("c")
    sid = lax.axis_index("s")
    wid = sid * NC + cid
    pltpu.sync_copy(ids2d.at[pl.ds(wid * krow, krow)], idx_v)
    cps = [
        pltpu.async_copy(table.at[idx_v.at[j]], rows_v.at[pl.ds(j * SUB, SUB)], sem)
        for j in range(krow)
    ]
    for cp in cps:
        cp.wait()
    pltpu.sync_copy(rows_v, out.at[pl.ds(wid * n_per_w, n_per_w)])


def _sc_gather(table, ids_pad):
    npad = ids_pad.shape[0]
    d = table.shape[1]
    n_per_w = npad // NW
    krow = n_per_w // SUB
    mesh = plsc.VectorSubcoreMesh(core_axis_name="c", subcore_axis_name="s")
    return pl.kernel(
        functools.partial(_gather_body, n_per_w, krow),
        out_type=jax.ShapeDtypeStruct((npad, d), jnp.float32),
        mesh=mesh,
        scratch_types=[
            pltpu.VMEM((krow, SUB), jnp.int32),
            pltpu.VMEM((n_per_w, d), jnp.float32),
            pltpu.SemaphoreType.DMA,
        ],
    )(table, ids_pad.reshape(npad // SUB, SUB))


def _edge_body(n, k, nch, hwf, gidx2, dst2, znd, zc8, ones8, agg_out, cnt_out,
               idx_v, dst_v, rows_v, ones_v, agg_sh, cnt_sh, gsem, ssem, csem):
    cid = lax.axis_index("c")
    sid = lax.axis_index("s")
    wid = sid * NC + cid
    seg = n // NS

    pltpu.sync_copy(znd.at[pl.ds(sid * seg, seg)], agg_sh.at[pl.ds(sid * seg, seg)])
    pltpu.sync_copy(zc8.at[pl.ds(sid * seg, seg)], cnt_sh.at[pl.ds(sid * seg, seg)])
    pltpu.sync_copy(ones8, ones_v)
    plsc.subcore_barrier()

    row_base = wid * (k * nch)

    def chunk(c, carry):
        row = row_base + c * k
        pltpu.sync_copy(gidx2.at[pl.ds(row, k)], idx_v)
        pltpu.sync_copy(dst2.at[pl.ds(row, k)], dst_v)
        gcps = [
            pltpu.async_copy(hwf.at[idx_v.at[j]], rows_v.at[pl.ds(j * SUB, SUB)], gsem)
            for j in range(k)
        ]
        for cp in gcps:
            cp.wait()
        scps = [
            pltpu.async_copy(rows_v.at[pl.ds(j * SUB, SUB)], agg_sh.at[dst_v.at[j]],
                             ssem, add=True)
            for j in range(k)
        ]
        ccps = [
            pltpu.async_copy(ones_v, cnt_sh.at[dst_v.at[j]], csem, add=True)
            for j in range(k)
        ]
        for cp in scps:
            cp.wait()
        for cp in ccps:
            cp.wait()
        return carry

    lax.fori_loop(0, nch, chunk, 0)
    plsc.subcore_barrier()

    pltpu.sync_copy(agg_sh.at[pl.ds(sid * seg, seg)],
                    agg_out.at[cid, pl.ds(sid * seg, seg)])
    pltpu.sync_copy(cnt_sh.at[pl.ds(sid * seg, seg)],
                    cnt_out.at[cid, pl.ds(sid * seg, seg)])


def _sc_edges(hwf, gidx, dst, n):
    e = gidx.shape[0]
    d = hwf.shape[1]
    rows_per_tile = e // (NW * SUB)
    k = 5
    assert e % (NW * SUB) == 0 and rows_per_tile % k == 0 and n % NS == 0
    nch = rows_per_tile // k
    znd = jnp.zeros((n, d), jnp.float32)
    zc8 = jnp.zeros((n, 8), jnp.float32)
    ones8 = jnp.ones((SUB, 8), jnp.float32)
    mesh = plsc.VectorSubcoreMesh(core_axis_name="c", subcore_axis_name="s")
    return pl.kernel(
        functools.partial(_edge_body, n, k, nch),
        out_type=[
            jax.ShapeDtypeStruct((NC, n, d), jnp.float32),
            jax.ShapeDtypeStruct((NC, n, 8), jnp.float32),
        ],
        mesh=mesh,
        scratch_types=[
            pltpu.VMEM((k, SUB), jnp.int32),
            pltpu.VMEM((k, SUB), jnp.int32),
            pltpu.VMEM((k * SUB, d), jnp.float32),
            pltpu.VMEM((SUB, 8), jnp.float32),
            pltpu.VMEM_SHARED((n, d), jnp.float32),
            pltpu.VMEM_SHARED((n, 8), jnp.float32),
            pltpu.SemaphoreType.DMA,
            pltpu.SemaphoreType.DMA,
            pltpu.SemaphoreType.DMA,
        ],
    )(hwf, gidx.reshape(e // SUB, SUB), dst.reshape(e // SUB, SUB),
      znd, zc8, ones8)


def _mm_body(h_ref, w_ref, o_ref):
    o_ref[0] = jnp.dot(h_ref[...], w_ref[0], preferred_element_type=jnp.float32)


def _tc_relation_matmuls(h, weight):
    npad, d = h.shape
    r = weight.shape[0]
    bn = 512
    return pl.pallas_call(
        _mm_body,
        grid=(r, npad // bn),
        in_specs=[
            pl.BlockSpec((bn, d), lambda ri, nb: (nb, 0)),
            pl.BlockSpec((1, d, d), lambda ri, nb: (ri, 0, 0)),
        ],
        out_specs=pl.BlockSpec((1, bn, d), lambda ri, nb: (ri, nb, 0)),
        out_shape=jax.ShapeDtypeStruct((r, npad, d), jnp.float32),
    )(h, weight)


def _fin_body(agg_ref, cnt_ref, w_ref, b_ref, o_ref):
    a = agg_ref[0] + agg_ref[1]
    c = cnt_ref[0, :, 0:1] + cnt_ref[1, :, 0:1]
    h = a / jnp.maximum(c, 1.0)
    y = jnp.dot(h, w_ref[...], preferred_element_type=jnp.float32) + b_ref[...]
    o_ref[...] = jax.nn.sigmoid(y)


def _tc_finish(agg_part, cnt_part, fc_w_pad, fc_b_pad):
    n, d = agg_part.shape[1], agg_part.shape[2]
    bn = 400
    return pl.pallas_call(
        _fin_body,
        grid=(n // bn,),
        in_specs=[
            pl.BlockSpec((2, bn, d), lambda nb: (0, nb, 0)),
            pl.BlockSpec((2, bn, 8), lambda nb: (0, nb, 0)),
            pl.BlockSpec((d, 128), lambda nb: (0, 0)),
            pl.BlockSpec((1, 128), lambda nb: (0, 0)),
        ],
        out_specs=pl.BlockSpec((bn, 128), lambda nb: (nb, 0)),
        out_shape=jax.ShapeDtypeStruct((n, 128), jnp.float32),
    )(agg_part, cnt_part, fc_w_pad, fc_b_pad)


def kernel(node_ids, edge_index, etype, entity, weight, fc_w, fc_b):
    n = node_ids.shape[0]
    d = entity.shape[1]
    t = fc_w.shape[1]

    npad = -(-n // (NW * SUB)) * (NW * SUB)
    ids_pad = jnp.pad(node_ids.astype(jnp.int32), (0, npad - n))
    h = _sc_gather(entity, ids_pad)

    hw = _tc_relation_matmuls(h, weight)
    hwf = hw.reshape(weight.shape[0] * npad, d)

    src = edge_index[0].astype(jnp.int32)
    dst = edge_index[1].astype(jnp.int32)
    gidx = etype.astype(jnp.int32) * npad + src
    agg_part, cnt_part = _sc_edges(hwf, gidx, dst, n)

    fc_w_pad = jnp.pad(fc_w, ((0, 0), (0, 128 - t)))
    fc_b_pad = jnp.pad(fc_b, (0, 128 - t)).reshape(1, 128)
    out_pad = _tc_finish(agg_part, cnt_part, fc_w_pad, fc_b_pad)
    return out_pad[:, :t]


# speedup vs baseline: 9.2674x; 9.2674x over previous
import functools

import jax
import jax.numpy as jnp
from jax import lax
from jax.experimental import pallas as pl
from jax.experimental.pallas import tpu as pltpu
from jax.experimental.pallas import tpu_sc as plsc

_INFO = plsc.get_sparse_core_info()
NC = _INFO.num_cores
NS = _INFO.num_subcores
NW = NC * NS
SUB = 64
K = 8
HC = 256


def _gather_body(n_per_w, table, ids1d, out, idx_v, rows_v, sem):
    cid = lax.axis_index("c")
    sid = lax.axis_index("s")
    wid = sid * NC + cid
    pltpu.sync_copy(ids1d.at[pl.ds(wid * n_per_w, n_per_w)], idx_v)
    pltpu.async_copy(table.at[idx_v], rows_v, sem).wait()
    pltpu.sync_copy(rows_v, out.at[pl.ds(wid * n_per_w, n_per_w)])


def _sc_gather(table, ids_pad):
    npad = ids_pad.shape[0]
    d = table.shape[1]
    n_per_w = npad // NW
    mesh = plsc.VectorSubcoreMesh(core_axis_name="c", subcore_axis_name="s")
    return pl.kernel(
        functools.partial(_gather_body, n_per_w),
        out_type=jax.ShapeDtypeStruct((npad, d), jnp.float32),
        mesh=mesh,
        scratch_types=[
            pltpu.VMEM((n_per_w,), jnp.int32),
            pltpu.VMEM((n_per_w, d), jnp.float32),
            pltpu.SemaphoreType.DMA,
        ],
    )(table, ids_pad)


def _edge_body(npad, nch, d, hwf, gidx1, dst1, aridx, zrow, agg_out,
               idx_a, idx_b, dst_a, dst_b, rows_a, rows_b,
               agg_sh, gsem, ssem):
    cid = lax.axis_index("c")
    sid = lax.axis_index("s")
    wid = sid * NC + cid
    seg = npad // NS
    nq = seg // SUB

    pltpu.sync_copy(zrow, rows_a)
    for q in range(nq):
        pltpu.sync_copy(aridx.at[pl.ds(sid * seg + q * SUB, SUB)], dst_a)
        pltpu.async_copy(rows_a, agg_sh.at[dst_a], ssem).wait()
    plsc.subcore_barrier()

    def fire_gather(blk, g, buf, dbuf, ibuf):
        pltpu.sync_copy(dst1.at[pl.ds((blk * K + g) * SUB, SUB)], dbuf)
        pltpu.sync_copy(gidx1.at[pl.ds((blk * K + g) * SUB, SUB)], ibuf)
        return pltpu.async_copy(hwf.at[ibuf], buf, gsem)

    def chunk(c, carry):
        blk = wid * nch + c
        bufs = (rows_a, rows_b)
        dsts = (dst_a, dst_b)
        idxs = (idx_a, idx_b)
        cp = fire_gather(blk, 0, bufs[0], dsts[0], idxs[0])
        for g in range(K):
            cp.wait()
            if g + 1 < K:
                cp = fire_gather(blk, g + 1, bufs[(g + 1) % 2],
                                 dsts[(g + 1) % 2], idxs[(g + 1) % 2])
            pltpu.async_copy(bufs[g % 2], agg_sh.at[dsts[g % 2]],
                             ssem, add=True).wait()
        return carry

    lax.fori_loop(0, nch, chunk, 0)
    plsc.subcore_barrier()

    for q in range(nq):
        pltpu.sync_copy(aridx.at[pl.ds(sid * seg + q * SUB, SUB)], dst_a)
        pltpu.async_copy(agg_sh.at[dst_a], rows_a, gsem).wait()
        pltpu.sync_copy(rows_a, agg_out.at[cid, pl.ds(sid * seg + q * SUB, SUB)])


def _sc_edges(hwf, gidx, dst, npad):
    e = gidx.shape[0]
    d = hwf.shape[1]
    assert e % (NW * K * SUB) == 0 and npad % (NS * SUB) == 0
    nch = e // (NW * K * SUB)
    aridx = jnp.arange(npad, dtype=jnp.int32)
    zrow = jnp.zeros((SUB, d), jnp.float32)
    mesh = plsc.VectorSubcoreMesh(core_axis_name="c", subcore_axis_name="s")
    return pl.kernel(
        functools.partial(_edge_body, npad, nch, d),
        out_type=jax.ShapeDtypeStruct((NC, npad, d), jnp.float32),
        mesh=mesh,
        scratch_types=[
            pltpu.VMEM((SUB,), jnp.int32),
            pltpu.VMEM((SUB,), jnp.int32),
            pltpu.VMEM((SUB,), jnp.int32),
            pltpu.VMEM((SUB,), jnp.int32),
            pltpu.VMEM((SUB, d), jnp.float32),
            pltpu.VMEM((SUB, d), jnp.float32),
            pltpu.VMEM_SHARED((npad, d), jnp.float32),
            pltpu.SemaphoreType.DMA,
            pltpu.SemaphoreType.DMA,
        ],
    )(hwf, gidx, dst, aridx, zrow)


def _count_body(npad, epw, dst1, zhist, cnt_out, dbuf, hist):
    cid = lax.axis_index("c")
    sid = lax.axis_index("s")
    wid = sid * NC + cid
    pltpu.sync_copy(zhist, hist)
    pltpu.sync_copy(dst1.at[pl.ds(wid * epw, epw)], dbuf)
    one16 = jnp.ones((16,), jnp.float32)
    unroll = 8

    def grp(i, carry):
        base = i * (16 * unroll)
        for j in range(unroll):
            dv = dbuf[pl.ds(base + j * 16, 16)]
            rowv = lax.shift_right_logical(dv, HC.bit_length() - 1)
            colv = lax.bitwise_and(dv, HC - 1)
            plsc.addupdate_scatter(hist, [rowv, colv], one16)
        return carry

    lax.fori_loop(0, epw // (16 * unroll), grp, 0)
    pltpu.sync_copy(hist, cnt_out.at[wid])


def _sc_counts(dst, npad):
    e = dst.shape[0]
    epw = e // NW
    assert e % (NW * 16 * 8) == 0 and npad % HC == 0
    zhist = jnp.zeros((npad // HC, HC), jnp.float32)
    mesh = plsc.VectorSubcoreMesh(core_axis_name="c", subcore_axis_name="s")
    return pl.kernel(
        functools.partial(_count_body, npad, epw),
        out_type=jax.ShapeDtypeStruct((NW, npad // HC, HC), jnp.float32),
        mesh=mesh,
        compiler_params=pltpu.CompilerParams(needs_layout_passes=False),
        scratch_types=[
            pltpu.VMEM((epw,), jnp.int32),
            pltpu.VMEM((npad // HC, HC), jnp.float32),
        ],
    )(dst, zhist)


def _mm_body(h_ref, w_ref, o_ref):
    o_ref[0] = jnp.dot(h_ref[...], w_ref[0], preferred_element_type=jnp.float32)


def _tc_relation_matmuls(h, weight):
    npad, d = h.shape
    r = weight.shape[0]
    bn = 512
    return pl.pallas_call(
        _mm_body,
        grid=(r, npad // bn),
        in_specs=[
            pl.BlockSpec((bn, d), lambda ri, nb: (nb, 0)),
            pl.BlockSpec((1, d, d), lambda ri, nb: (ri, 0, 0)),
        ],
        out_specs=pl.BlockSpec((1, bn, d), lambda ri, nb: (ri, nb, 0)),
        out_shape=jax.ShapeDtypeStruct((r, npad, d), jnp.float32),
    )(h, weight)


def _fin_body(agg_ref, cnt_ref, w_ref, b_ref, o_ref):
    a = agg_ref[0] + agg_ref[1]
    c = jnp.sum(cnt_ref[...], axis=1, keepdims=True)
    h = a / jnp.maximum(c, 1.0)
    y = jnp.dot(h, w_ref[...], preferred_element_type=jnp.float32) + b_ref[...]
    o_ref[...] = jax.nn.sigmoid(y)


def _tc_finish(agg_part, cnt_nw, fc_w_pad, fc_b_pad, n):
    d = agg_part.shape[2]
    bn = 400
    return pl.pallas_call(
        _fin_body,
        grid=(n // bn,),
        in_specs=[
            pl.BlockSpec((2, bn, d), lambda nb: (0, nb, 0)),
            pl.BlockSpec((bn, NW), lambda nb: (nb, 0)),
            pl.BlockSpec((d, 128), lambda nb: (0, 0)),
            pl.BlockSpec((1, 128), lambda nb: (0, 0)),
        ],
        out_specs=pl.BlockSpec((bn, 128), lambda nb: (nb, 0)),
        out_shape=jax.ShapeDtypeStruct((n, 128), jnp.float32),
    )(agg_part, cnt_nw, fc_w_pad, fc_b_pad)


def kernel(node_ids, edge_index, etype, entity, weight, fc_w, fc_b):
    n = node_ids.shape[0]
    t = fc_w.shape[1]
    e = etype.shape[0]

    npad = -(-n // (NW * SUB)) * (NW * SUB)
    assert n < npad
    ids_pad = jnp.pad(node_ids.astype(jnp.int32), (0, npad - n))
    h = _sc_gather(entity, ids_pad)

    hw = _tc_relation_matmuls(h, weight)
    hwf = hw.reshape(weight.shape[0] * npad, h.shape[1])

    egrp = NW * K * SUB
    epad = -(-e // egrp) * egrp
    src = edge_index[0].astype(jnp.int32)
    dst = edge_index[1].astype(jnp.int32)
    gidx = jnp.pad(etype.astype(jnp.int32) * npad + src, (0, epad - e))
    dstp = jnp.pad(dst, (0, epad - e), constant_values=npad - 1)
    agg_part = _sc_edges(hwf, gidx, dstp, npad)
    cnt_part = _sc_counts(dstp, npad)

    cnt_nw = jnp.transpose(cnt_part.reshape(NW, npad))
    fc_w_pad = jnp.pad(fc_w, ((0, 0), (0, 128 - t)))
    fc_b_pad = jnp.pad(fc_b, (0, 128 - t)).reshape(1, 128)
    out_pad = _tc_finish(agg_part, cnt_nw, fc_w_pad, fc_b_pad, n)
    return out_pad[:, :t]


# speedup vs baseline: 9.2789x; 1.0012x over previous
import functools

import jax
import jax.numpy as jnp
from jax import lax
from jax.experimental import pallas as pl
from jax.experimental.pallas import tpu as pltpu
from jax.experimental.pallas import tpu_sc as plsc

_INFO = plsc.get_sparse_core_info()
NC = _INFO.num_cores
NS = _INFO.num_subcores
NW = NC * NS
SUB = 64
K = 8
HC = 256


def _gather_body(n_per_w, table, ids1d, out, idx_v, rows_v, sem):
    cid = lax.axis_index("c")
    sid = lax.axis_index("s")
    wid = sid * NC + cid
    pltpu.sync_copy(ids1d.at[pl.ds(wid * n_per_w, n_per_w)], idx_v)
    pltpu.async_copy(table.at[idx_v], rows_v, sem).wait()
    pltpu.sync_copy(rows_v, out.at[pl.ds(wid * n_per_w, n_per_w)])


def _sc_gather(table, ids_pad):
    npad = ids_pad.shape[0]
    d = table.shape[1]
    n_per_w = npad // NW
    mesh = plsc.VectorSubcoreMesh(core_axis_name="c", subcore_axis_name="s")
    return pl.kernel(
        functools.partial(_gather_body, n_per_w),
        out_type=jax.ShapeDtypeStruct((npad, d), jnp.float32),
        mesh=mesh,
        scratch_types=[
            pltpu.VMEM((n_per_w,), jnp.int32),
            pltpu.VMEM((n_per_w, d), jnp.float32),
            pltpu.SemaphoreType.DMA,
        ],
    )(table, ids_pad)


def _edge_body(npad, nch, d, hwf, gidx1, dst1, aridx, zrow, agg_out,
               idx_a, idx_b, dst_a, dst_b, rows_a, rows_b,
               agg_sh, gsem, ssem):
    cid = lax.axis_index("c")
    sid = lax.axis_index("s")
    wid = sid * NC + cid
    seg = npad // NS
    nq = seg // SUB

    pltpu.sync_copy(zrow, rows_a)
    for q in range(nq):
        pltpu.sync_copy(aridx.at[pl.ds(sid * seg + q * SUB, SUB)], dst_a)
        pltpu.async_copy(rows_a, agg_sh.at[dst_a], ssem).wait()
    plsc.subcore_barrier()

    def fire_gather(blk, g, buf, dbuf, ibuf):
        pltpu.sync_copy(dst1.at[pl.ds((blk * K + g) * SUB, SUB)], dbuf)
        pltpu.sync_copy(gidx1.at[pl.ds((blk * K + g) * SUB, SUB)], ibuf)
        return pltpu.async_copy(hwf.at[ibuf], buf, gsem)

    def chunk(c, carry):
        blk = wid * nch + c
        bufs = (rows_a, rows_b)
        dsts = (dst_a, dst_b)
        idxs = (idx_a, idx_b)
        cp = fire_gather(blk, 0, bufs[0], dsts[0], idxs[0])
        sprev = None
        for g in range(K):
            cp.wait()
            if sprev is not None:
                sprev.wait()
            if g + 1 < K:
                cp = fire_gather(blk, g + 1, bufs[(g + 1) % 2],
                                 dsts[(g + 1) % 2], idxs[(g + 1) % 2])
            sprev = pltpu.async_copy(bufs[g % 2], agg_sh.at[dsts[g % 2]],
                                     ssem, add=True)
        sprev.wait()
        return carry

    lax.fori_loop(0, nch, chunk, 0)
    plsc.subcore_barrier()

    for q in range(nq):
        pltpu.sync_copy(aridx.at[pl.ds(sid * seg + q * SUB, SUB)], dst_a)
        pltpu.async_copy(agg_sh.at[dst_a], rows_a, gsem).wait()
        pltpu.sync_copy(rows_a, agg_out.at[cid, pl.ds(sid * seg + q * SUB, SUB)])


def _sc_edges(hwf, gidx, dst, npad):
    e = gidx.shape[0]
    d = hwf.shape[1]
    assert e % (NW * K * SUB) == 0 and npad % (NS * SUB) == 0
    nch = e // (NW * K * SUB)
    aridx = jnp.arange(npad, dtype=jnp.int32)
    zrow = jnp.zeros((SUB, d), jnp.float32)
    mesh = plsc.VectorSubcoreMesh(core_axis_name="c", subcore_axis_name="s")
    return pl.kernel(
        functools.partial(_edge_body, npad, nch, d),
        out_type=jax.ShapeDtypeStruct((NC, npad, d), jnp.float32),
        mesh=mesh,
        scratch_types=[
            pltpu.VMEM((SUB,), jnp.int32),
            pltpu.VMEM((SUB,), jnp.int32),
            pltpu.VMEM((SUB,), jnp.int32),
            pltpu.VMEM((SUB,), jnp.int32),
            pltpu.VMEM((SUB, d), jnp.float32),
            pltpu.VMEM((SUB, d), jnp.float32),
            pltpu.VMEM_SHARED((npad, d), jnp.float32),
            pltpu.SemaphoreType.DMA,
            pltpu.SemaphoreType.DMA,
        ],
    )(hwf, gidx, dst, aridx, zrow)


def _count_body(npad, epw, dst1, zhist, cnt_out, dbuf, hist):
    cid = lax.axis_index("c")
    sid = lax.axis_index("s")
    wid = sid * NC + cid
    pltpu.sync_copy(zhist, hist)
    pltpu.sync_copy(dst1.at[pl.ds(wid * epw, epw)], dbuf)
    one16 = jnp.ones((16,), jnp.float32)
    unroll = 8

    def grp(i, carry):
        base = i * (16 * unroll)
        for j in range(unroll):
            dv = dbuf[pl.ds(base + j * 16, 16)]
            rowv = lax.shift_right_logical(dv, HC.bit_length() - 1)
            colv = lax.bitwise_and(dv, HC - 1)
            plsc.addupdate_scatter(hist, [rowv, colv], one16)
        return carry

    lax.fori_loop(0, epw // (16 * unroll), grp, 0)
    pltpu.sync_copy(hist, cnt_out.at[wid])


def _sc_counts(dst, npad):
    e = dst.shape[0]
    epw = e // NW
    assert e % (NW * 16 * 8) == 0 and npad % HC == 0
    zhist = jnp.zeros((npad // HC, HC), jnp.float32)
    mesh = plsc.VectorSubcoreMesh(core_axis_name="c", subcore_axis_name="s")
    return pl.kernel(
        functools.partial(_count_body, npad, epw),
        out_type=jax.ShapeDtypeStruct((NW, npad // HC, HC), jnp.float32),
        mesh=mesh,
        compiler_params=pltpu.CompilerParams(needs_layout_passes=False),
        scratch_types=[
            pltpu.VMEM((epw,), jnp.int32),
            pltpu.VMEM((npad // HC, HC), jnp.float32),
        ],
    )(dst, zhist)


def _mm_body(h_ref, w_ref, o_ref):
    o_ref[0] = jnp.dot(h_ref[...], w_ref[0], preferred_element_type=jnp.float32)


def _tc_relation_matmuls(h, weight):
    npad, d = h.shape
    r = weight.shape[0]
    bn = 512
    return pl.pallas_call(
        _mm_body,
        grid=(r, npad // bn),
        in_specs=[
            pl.BlockSpec((bn, d), lambda ri, nb: (nb, 0)),
            pl.BlockSpec((1, d, d), lambda ri, nb: (ri, 0, 0)),
        ],
        out_specs=pl.BlockSpec((1, bn, d), lambda ri, nb: (ri, nb, 0)),
        out_shape=jax.ShapeDtypeStruct((r, npad, d), jnp.float32),
    )(h, weight)


def _fin_body(agg_ref, cnt_ref, w_ref, b_ref, o_ref):
    a = agg_ref[0] + agg_ref[1]
    c = jnp.sum(cnt_ref[...], axis=1, keepdims=True)
    h = a / jnp.maximum(c, 1.0)
    y = jnp.dot(h, w_ref[...], preferred_element_type=jnp.float32) + b_ref[...]
    o_ref[...] = jax.nn.sigmoid(y)


def _tc_finish(agg_part, cnt_nw, fc_w_pad, fc_b_pad, n):
    d = agg_part.shape[2]
    bn = 400
    return pl.pallas_call(
        _fin_body,
        grid=(n // bn,),
        in_specs=[
            pl.BlockSpec((2, bn, d), lambda nb: (0, nb, 0)),
            pl.BlockSpec((bn, NW), lambda nb: (nb, 0)),
            pl.BlockSpec((d, 128), lambda nb: (0, 0)),
            pl.BlockSpec((1, 128), lambda nb: (0, 0)),
        ],
        out_specs=pl.BlockSpec((bn, 128), lambda nb: (nb, 0)),
        out_shape=jax.ShapeDtypeStruct((n, 128), jnp.float32),
    )(agg_part, cnt_nw, fc_w_pad, fc_b_pad)


def kernel(node_ids, edge_index, etype, entity, weight, fc_w, fc_b):
    n = node_ids.shape[0]
    t = fc_w.shape[1]
    e = etype.shape[0]

    npad = -(-n // (NW * SUB)) * (NW * SUB)
    assert n < npad
    ids_pad = jnp.pad(node_ids.astype(jnp.int32), (0, npad - n))
    h = _sc_gather(entity, ids_pad)

    hw = _tc_relation_matmuls(h, weight)
    hwf = hw.reshape(weight.shape[0] * npad, h.shape[1])

    egrp = NW * K * SUB
    epad = -(-e // egrp) * egrp
    src = edge_index[0].astype(jnp.int32)
    dst = edge_index[1].astype(jnp.int32)
    gidx = jnp.pad(etype.astype(jnp.int32) * npad + src, (0, epad - e))
    dstp = jnp.pad(dst, (0, epad - e), constant_values=npad - 1)
    agg_part = _sc_edges(hwf, gidx, dstp, npad)
    cnt_part = _sc_counts(dstp, npad)

    cnt_nw = jnp.transpose(cnt_part.reshape(NW, npad))
    fc_w_pad = jnp.pad(fc_w, ((0, 0), (0, 128 - t)))
    fc_b_pad = jnp.pad(fc_b, (0, 128 - t)).reshape(1, 128)
    out_pad = _tc_finish(agg_part, cnt_nw, fc_w_pad, fc_b_pad, n)
    return out_pad[:, :t]
